# Initial kernel scaffold; baseline (speedup 1.0000x reference)
import jax
import jax.numpy as jnp
from jax.experimental import pallas as pl


def kernel(x, std, weights):
    raise NotImplementedError("write your pallas kernel here")


# speedup vs baseline: 8.7539x; 8.7539x over previous
import jax
import jax.numpy as jnp
from jax.experimental import pallas as pl
from jax.experimental.pallas import tpu as pltpu


def _tm_kernel(side, inv_ref, x_ref, w_ref, out_ref):
    x = x_ref[:]
    w = w_ref[:]
    inv = inv_ref[0, 0]

    xw = jax.lax.dot_general(
        x, w, (((1,), (0,)), ((), ())),
        precision=jax.lax.Precision.HIGHEST,
        preferred_element_type=jnp.float32,
    )
    x2 = jnp.sum(x * x, axis=1, keepdims=True)
    w2 = jnp.sum(w * w, axis=0, keepdims=True)
    n2 = x2 - 2.0 * xw + w2

    mn = jnp.min(n2, axis=1, keepdims=True)
    colid = jax.lax.broadcasted_iota(jnp.int32, n2.shape, 1)
    idx = jnp.min(jnp.where(n2 == mn, colid, n2.shape[1]), axis=1,
                  keepdims=True)

    rowf = (idx // side).astype(jnp.float32)
    colf = (idx % side).astype(jnp.float32)
    gr = (colid // side).astype(jnp.float32)
    gc = (colid % side).astype(jnp.float32)
    dr = gr - rowf
    dc = gc - colf
    phi = jnp.exp(-inv * (dr * dr + dc * dc))
    denom = jnp.sum(phi, axis=1, keepdims=True)
    out_ref[:] = n2 * (phi / denom)


def kernel(x, std, weights):
    B, D = x.shape
    O = weights.shape[1]
    side = int(round(float(O) ** 0.5))
    BB = 256 if B % 256 == 0 else B

    std_f = jnp.asarray(std).astype(jnp.float32)
    inv = (0.5 * std_f ** (-2)).reshape(1, 1)

    import functools
    body = functools.partial(_tm_kernel, side)
    return pl.pallas_call(
        body,
        grid=(B // BB,),
        in_specs=[
            pl.BlockSpec(memory_space=pltpu.SMEM),
            pl.BlockSpec((BB, D), lambda i: (i, 0)),
            pl.BlockSpec((D, O), lambda i: (0, 0)),
        ],
        out_specs=pl.BlockSpec((BB, O), lambda i: (i, 0)),
        out_shape=jax.ShapeDtypeStruct((B, O), jnp.float32),
    )(inv, x, weights)
